# baseline (speedup 1.0000x reference)
import functools

import jax
import jax.numpy as jnp
from jax import lax
from jax.experimental import pallas as pl
from jax.experimental.pallas import tpu as pltpu, tpu_sc as plsc

N = 10000
H = 128
EPS = 1e-12

NC = 2
NS = 16
K = 128
C = 32

ROW_BLK = 1000


def _h0_body(x_ref, w_ref, b_ref, o_ref):
    y = lax.dot_general(x_ref[...], w_ref[...], (((1,), (1,)), ((), ())),
                        preferred_element_type=jnp.float32)
    y = y + b_ref[...]
    nrm = jnp.sqrt(jnp.sum(y * y, axis=1, keepdims=True))
    o_ref[...] = y / jnp.maximum(nrm, EPS)


def _combine_body(ppi_ref, res_ref, w_ref, b_ref, o_ref):
    y = lax.dot_general(ppi_ref[...], w_ref[...], (((1,), (1,)), ((), ())),
                        preferred_element_type=jnp.float32)
    o_ref[...] = jnp.maximum(y + b_ref[...], 0.0) + res_ref[...]


def _final_body(h_ref, w_ref, b_ref, o_ref):
    y = lax.dot_general(h_ref[...], w_ref[...], (((1,), (1,)), ((), ())),
                        preferred_element_type=jnp.float32)
    o_ref[...] = y + b_ref[...]


def _row_grid(n):
    return (n // ROW_BLK,)


def _tc_h0(x, w, b):
    return pl.pallas_call(
        _h0_body,
        grid=_row_grid(N),
        in_specs=[
            pl.BlockSpec((ROW_BLK, x.shape[1]), lambda i: (i, 0)),
            pl.BlockSpec(w.shape, lambda i: (0, 0)),
            pl.BlockSpec((1, H), lambda i: (0, 0)),
        ],
        out_specs=pl.BlockSpec((ROW_BLK, H), lambda i: (i, 0)),
        out_shape=jax.ShapeDtypeStruct((N, H), jnp.float32),
    )(x, w, b)


def _tc_combine(ppi, res, w, b):
    return pl.pallas_call(
        _combine_body,
        grid=_row_grid(N),
        in_specs=[
            pl.BlockSpec((ROW_BLK, H), lambda i: (i, 0)),
            pl.BlockSpec((ROW_BLK, H), lambda i: (i, 0)),
            pl.BlockSpec((H, H), lambda i: (0, 0)),
            pl.BlockSpec((1, H), lambda i: (0, 0)),
        ],
        out_specs=pl.BlockSpec((ROW_BLK, H), lambda i: (i, 0)),
        out_shape=jax.ShapeDtypeStruct((N, H), jnp.float32),
    )(ppi, res, w, b)


def _tc_final(h, w, b):
    l = w.shape[0]
    return pl.pallas_call(
        _final_body,
        grid=_row_grid(N),
        in_specs=[
            pl.BlockSpec((ROW_BLK, H), lambda i: (i, 0)),
            pl.BlockSpec((l, H), lambda i: (0, 0)),
            pl.BlockSpec((1, l), lambda i: (0, 0)),
        ],
        out_specs=pl.BlockSpec((ROW_BLK, l), lambda i: (i, 0)),
        out_shape=jax.ShapeDtypeStruct((N, l), jnp.float32),
    )(h, w, b)


def _sc_segment_sums(h, src3, dst3, w24, zeros, nb):
    row_stride, row_span = 624, 640

    mesh = plsc.VectorSubcoreMesh(core_axis_name="c", subcore_axis_name="s",
                                  num_cores=NC, num_subcores=NS)

    @functools.partial(
        pl.kernel,
        mesh=mesh,
        out_type=jax.ShapeDtypeStruct((NC, N, H), jnp.float32),
        scratch_types=[
            pltpu.VMEM_SHARED((N, H), jnp.float32),
            pltpu.VMEM((C, K), jnp.int32),
            pltpu.VMEM((C, K), jnp.int32),
            pltpu.VMEM((C, K), jnp.float32),
            pltpu.VMEM((K, H), jnp.float32),
            pltpu.VMEM((K, H), jnp.float32),
            pltpu.SemaphoreType.DMA,
            pltpu.SemaphoreType.DMA,
        ],
    )
    def sc_kernel(h_hbm, src_hbm, dst_hbm, w2_hbm, z_hbm, out_hbm,
                  acc, src_v, dst_v, w_v, rows0, rows1, sem0, sem1):
        c = lax.axis_index("c")
        s = lax.axis_index("s")
        rows = (rows0, rows1)
        sems = (sem0, sem1)

        pltpu.sync_copy(z_hbm, rows0)
        for z in range(row_span // K):
            pltpu.sync_copy(rows0,
                            acc.at[pl.ds(s * row_stride + z * K, K)])
        plsc.subcore_barrier()

        def issue(rb, par):
            pltpu.async_copy(h_hbm.at[src_v.at[rb]], rows[par], sems[par])

        def half(rb, par):
            pltpu.make_async_copy(h_hbm.at[src_v.at[rb]], rows[par],
                                  sems[par]).wait()

            def group_body(g, carry):
                w16 = w_v[rb, pl.ds(g * 16, 16)]
                for j in range(16):
                    e = g * 16 + j
                    wb = w16[j]
                    for ch in range(H // 16):
                        sl = pl.ds(ch * 16, 16)
                        rows[par][e, sl] = rows[par][e, sl] * wb
                return carry

            lax.fori_loop(0, K // 16, group_body, 0)
            pltpu.sync_copy(rows[par], acc.at[dst_v.at[rb]], add=True)

            @pl.when(rb + 2 < C)
            def _():
                issue(rb + 2, par)

        def chunk_body(ch, carry):
            sl = pl.ds(ch * C, C)
            pltpu.sync_copy(src_hbm.at[s, sl], src_v)
            pltpu.sync_copy(dst_hbm.at[s, sl], dst_v)
            pltpu.sync_copy(w2_hbm.at[c, s, sl], w_v)
            issue(0, 0)
            issue(1, 1)

            def loop_body(i, carry2):
                half(2 * i, 0)
                half(2 * i + 1, 1)
                return carry2

            lax.fori_loop(0, C // 2, loop_body, 0)
            return carry

        lax.fori_loop(0, nb // C, chunk_body, 0)
        plsc.subcore_barrier()

        pltpu.sync_copy(acc.at[pl.ds(s * row_stride, row_span)],
                        out_hbm.at[c, pl.ds(s * row_stride, row_span)])

    return sc_kernel(h, src3, dst3, w24, zeros)


def kernel(inputs, edge_index, edge_ppi, edge_self, W_in, b_in, input_bias,
           W_ppi1, b_ppi1, W_ppi2, b_ppi2, W_out, b_out):
    e = edge_index.shape[1]
    blk = NS * K * C
    e_pad = ((e + blk - 1) // blk) * blk
    pad = e_pad - e
    nb = e_pad // (NS * K)

    src = jnp.concatenate([edge_index[0], jnp.zeros((pad,), jnp.int32)])
    dst = jnp.concatenate([edge_index[1], jnp.zeros((pad,), jnp.int32)])
    wpad = jnp.zeros((pad,), jnp.float32)
    w2 = jnp.stack([jnp.concatenate([edge_self, wpad]),
                    jnp.concatenate([edge_ppi, wpad])])
    src3 = src.reshape(NS, nb, K)
    dst3 = dst.reshape(NS, nb, K)
    w24 = w2.reshape(2, NS, nb, K)
    zeros = jnp.zeros((K, H), jnp.float32)

    bias0 = (b_in + input_bias).reshape(1, H)
    h = _tc_h0(inputs, W_in, bias0)

    for w, b in ((W_ppi1, b_ppi1), (W_ppi2, b_ppi2)):
        sums = _sc_segment_sums(h, src3, dst3, w24, zeros, nb)
        h = _tc_combine(sums[1], sums[0], w, b.reshape(1, H))

    return _tc_final(h, W_out, b_out.reshape(1, W_out.shape[0]))
